# Initial kernel scaffold; baseline (speedup 1.0000x reference)
import jax
import jax.numpy as jnp
from jax.experimental import pallas as pl


def kernel(ids_0, ids_1, W0, W1):
    raise NotImplementedError("write your pallas kernel here")


# speedup vs baseline: 19.6083x; 19.6083x over previous
import functools

import jax
import jax.numpy as jnp
from jax import lax
from jax.experimental import pallas as pl
from jax.experimental.pallas import tpu as pltpu
from jax.experimental.pallas import tpu_sc as plsc

ZCH0 = 16
ZCH1 = 32
EMBED = 64
BATCH = 4096
HIST = 20
N = BATCH * HIST

NC = 2
NS = 16
LANES = 16
NW = NC * NS
CHUNK = N // NW
VECS = CHUNK // LANES

_SC_MESH = plsc.VectorSubcoreMesh(core_axis_name="c", subcore_axis_name="s")


@functools.partial(
    pl.kernel,
    out_type=jax.ShapeDtypeStruct((NW, LANES), jnp.float32),
    mesh=_SC_MESH,
    scratch_types=[
        pltpu.VMEM((CHUNK,), jnp.int32),
        pltpu.VMEM((CHUNK,), jnp.int32),
        pltpu.VMEM((ZCH0 * EMBED,), jnp.float32),
        pltpu.VMEM((ZCH1 * EMBED,), jnp.float32),
        pltpu.VMEM((ZCH1,), jnp.float32),
        pltpu.VMEM((LANES,), jnp.float32),
    ],
    compiler_params=pltpu.CompilerParams(needs_layout_passes=False),
)
def _sc_partials(ids0_hbm, ids1_hbm, w0_hbm, w1_hbm, out_hbm,
                 ids0_v, ids1_v, w0_v, w1_v, rs1_v, acc_v):
    wid = lax.axis_index("s") * NC + lax.axis_index("c")
    base = wid * CHUNK
    pltpu.sync_copy(ids0_hbm.at[pl.ds(base, CHUNK)], ids0_v)
    pltpu.sync_copy(ids1_hbm.at[pl.ds(base, CHUNK)], ids1_v)
    pltpu.sync_copy(w0_hbm, w0_v)
    pltpu.sync_copy(w1_hbm, w1_v)

    lane = lax.iota(jnp.int32, LANES)

    row_base = lane * EMBED
    rs0 = jnp.zeros((LANES,), jnp.float32)
    for dcol in range(EMBED):
        rs0 = rs0 + plsc.load_gather(w0_v, [row_base + dcol])

    for half in range(2):
        row_base1 = (lane + half * LANES) * EMBED
        rs1 = jnp.zeros((LANES,), jnp.float32)
        for dcol in range(EMBED):
            rs1 = rs1 + plsc.load_gather(w1_v, [row_base1 + dcol])
        rs1_v[pl.ds(half * LANES, LANES)] = rs1

    acc_v[...] = rs0

    def body0(i, acc):
        ids = ids0_v[pl.ds(pl.multiple_of(i * LANES, LANES), LANES)]
        return acc + plsc.load_gather(acc_v, [lax.bitwise_and(ids, ZCH0 - 1)])

    def body1(i, acc):
        ids = ids1_v[pl.ds(pl.multiple_of(i * LANES, LANES), LANES)]
        return acc + plsc.load_gather(rs1_v, [lax.bitwise_and(ids, ZCH1 - 1)])

    acc = lax.fori_loop(0, VECS, body0, jnp.zeros((LANES,), jnp.float32))
    acc = lax.fori_loop(0, VECS, body1, acc)
    acc_v[...] = acc
    pltpu.sync_copy(acc_v, out_hbm.at[wid])


def _tc_finish(partials):
    def body(p_ref, o_ref):
        o_ref[0, 0] = jnp.sum(p_ref[...]) * (1.0 / (BATCH * 2 * EMBED))

    return pl.pallas_call(
        body,
        out_shape=jax.ShapeDtypeStruct((1, 1), jnp.float32),
        out_specs=pl.BlockSpec(memory_space=pltpu.SMEM),
    )(partials)


def kernel(ids_0, ids_1, W0, W1):
    partials = _sc_partials(
        ids_0.reshape(-1), ids_1.reshape(-1), W0.reshape(-1), W1.reshape(-1))
    return _tc_finish(partials)[0, 0]


# speedup vs baseline: 20.4983x; 1.0454x over previous
import functools

import jax
import jax.numpy as jnp
from jax import lax
from jax.experimental import pallas as pl
from jax.experimental.pallas import tpu as pltpu
from jax.experimental.pallas import tpu_sc as plsc

ZCH0 = 16
ZCH1 = 32
EMBED = 64
BATCH = 4096
HIST = 20
N = BATCH * HIST

NC = 2
NS = 16
LANES = 16
NW = NC * NS
CHUNK = N // NW
VECS = CHUNK // LANES

_SC_MESH = plsc.VectorSubcoreMesh(core_axis_name="c", subcore_axis_name="s")


@functools.partial(
    pl.kernel,
    out_type=jax.ShapeDtypeStruct((NW, LANES), jnp.float32),
    mesh=_SC_MESH,
    scratch_types=[
        pltpu.VMEM((CHUNK,), jnp.int32),
        pltpu.VMEM((CHUNK,), jnp.int32),
        pltpu.VMEM((ZCH0 * EMBED,), jnp.float32),
        pltpu.VMEM((ZCH1 * EMBED,), jnp.float32),
        pltpu.VMEM((ZCH1,), jnp.float32),
        pltpu.VMEM((LANES,), jnp.float32),
        pltpu.SemaphoreType.DMA,
        pltpu.SemaphoreType.DMA,
        pltpu.SemaphoreType.DMA,
    ],
    compiler_params=pltpu.CompilerParams(needs_layout_passes=False),
)
def _sc_partials(ids0_hbm, ids1_hbm, w0_hbm, w1_hbm, out_hbm,
                 ids0_v, ids1_v, w0_v, w1_v, rs1_v, acc_v,
                 sem0, sem1, semw):
    wid = lax.axis_index("s") * NC + lax.axis_index("c")
    base = wid * CHUNK
    cp0 = pltpu.async_copy(ids0_hbm.at[pl.ds(base, CHUNK)], ids0_v, sem0)
    cp1 = pltpu.async_copy(ids1_hbm.at[pl.ds(base, CHUNK)], ids1_v, sem1)
    cpw0 = pltpu.async_copy(w0_hbm, w0_v, semw)
    cpw1 = pltpu.async_copy(w1_hbm, w1_v, semw)

    lane = lax.iota(jnp.int32, LANES)
    cpw0.wait()
    cpw1.wait()

    row_base = lane * EMBED
    parts = [jnp.zeros((LANES,), jnp.float32) for _ in range(4)]
    for dcol in range(EMBED):
        parts[dcol % 4] = parts[dcol % 4] + plsc.load_gather(
            w0_v, [row_base + dcol])
    rs0 = (parts[0] + parts[1]) + (parts[2] + parts[3])

    for half in range(2):
        row_base1 = (lane + half * LANES) * EMBED
        parts = [jnp.zeros((LANES,), jnp.float32) for _ in range(4)]
        for dcol in range(EMBED):
            parts[dcol % 4] = parts[dcol % 4] + plsc.load_gather(
                w1_v, [row_base1 + dcol])
        rs1_v[pl.ds(half * LANES, LANES)] = (parts[0] + parts[1]) + (
            parts[2] + parts[3])

    acc_v[...] = rs0

    UNROLL = 8

    def body0(i, accs):
        accs = list(accs)
        for u in range(UNROLL):
            j = i * UNROLL + u
            ids = ids0_v[pl.ds(pl.multiple_of(j * LANES, LANES), LANES)]
            accs[u % 2] = accs[u % 2] + plsc.load_gather(
                acc_v, [lax.bitwise_and(ids, ZCH0 - 1)])
        return tuple(accs)

    def body1(i, accs):
        accs = list(accs)
        for u in range(UNROLL):
            j = i * UNROLL + u
            ids = ids1_v[pl.ds(pl.multiple_of(j * LANES, LANES), LANES)]
            accs[u % 2] = accs[u % 2] + plsc.load_gather(
                rs1_v, [lax.bitwise_and(ids, ZCH1 - 1)])
        return tuple(accs)

    zero2 = (jnp.zeros((LANES,), jnp.float32),) * 2
    cp0.wait()
    a0, b0 = lax.fori_loop(0, VECS // UNROLL, body0, zero2)
    cp1.wait()
    a1, b1 = lax.fori_loop(0, VECS // UNROLL, body1, zero2)
    acc_v[...] = (a0 + b0) + (a1 + b1)
    pltpu.sync_copy(acc_v, out_hbm.at[wid])


def _tc_finish(partials):
    def body(p_ref, o_ref):
        o_ref[0, 0] = jnp.sum(p_ref[...]) * (1.0 / (BATCH * 2 * EMBED))

    return pl.pallas_call(
        body,
        out_shape=jax.ShapeDtypeStruct((1, 1), jnp.float32),
        out_specs=pl.BlockSpec(memory_space=pltpu.SMEM),
    )(partials)


def kernel(ids_0, ids_1, W0, W1):
    partials = _sc_partials(
        ids_0.reshape(-1), ids_1.reshape(-1), W0.reshape(-1), W1.reshape(-1))
    return _tc_finish(partials)[0, 0]


# speedup vs baseline: 22.5132x; 1.0983x over previous
import functools

import jax
import jax.numpy as jnp
from jax import lax
from jax.experimental import pallas as pl
from jax.experimental.pallas import tpu as pltpu
from jax.experimental.pallas import tpu_sc as plsc

ZCH0 = 16
ZCH1 = 32
EMBED = 64
BATCH = 4096
HIST = 20
N = BATCH * HIST

NC = 2
NS = 16
LANES = 16
NW = NC * NS
ROWS = BATCH // NW
CHUNK = ROWS * HIST
VECS = CHUNK // LANES

_SC_MESH = plsc.VectorSubcoreMesh(core_axis_name="c", subcore_axis_name="s")


@functools.partial(
    pl.kernel,
    out_type=jax.ShapeDtypeStruct((NW, LANES), jnp.float32),
    mesh=_SC_MESH,
    scratch_types=[
        pltpu.VMEM((ROWS, HIST), jnp.int32),
        pltpu.VMEM((ROWS, HIST), jnp.int32),
        pltpu.VMEM((ZCH0, EMBED), jnp.float32),
        pltpu.VMEM((ZCH1, EMBED), jnp.float32),
        pltpu.VMEM((ZCH1,), jnp.float32),
        pltpu.VMEM((LANES,), jnp.float32),
        pltpu.SemaphoreType.DMA,
        pltpu.SemaphoreType.DMA,
        pltpu.SemaphoreType.DMA,
    ],
    compiler_params=pltpu.CompilerParams(needs_layout_passes=False),
)
def _sc_partials(ids0_hbm, ids1_hbm, w0_hbm, w1_hbm, out_hbm,
                 ids0_v, ids1_v, w0_v, w1_v, rs1_v, acc_v,
                 sem0, sem1, semw):
    wid = lax.axis_index("s") * NC + lax.axis_index("c")
    rbase = wid * ROWS
    cp0 = pltpu.async_copy(ids0_hbm.at[pl.ds(rbase, ROWS), :], ids0_v, sem0)
    cp1 = pltpu.async_copy(ids1_hbm.at[pl.ds(rbase, ROWS), :], ids1_v, sem1)
    cpw0 = pltpu.async_copy(w0_hbm, w0_v, semw)
    cpw1 = pltpu.async_copy(w1_hbm, w1_v, semw)

    lane = lax.iota(jnp.int32, LANES)
    cpw0.wait()
    cpw1.wait()

    def rowsums(w_ref, row0):
        rows = lane + row0
        parts = [jnp.zeros((LANES,), jnp.float32) for _ in range(4)]
        for dcol in range(EMBED):
            col = jnp.full((LANES,), dcol, jnp.int32)
            parts[dcol % 4] = parts[dcol % 4] + plsc.load_gather(
                w_ref, [rows, col])
        return (parts[0] + parts[1]) + (parts[2] + parts[3])

    rs0 = rowsums(w0_v, 0)
    rs1_v[pl.ds(0, LANES)] = rowsums(w1_v, 0)
    rs1_v[pl.ds(LANES, LANES)] = rowsums(w1_v, LANES)

    acc_v[...] = rs0

    UNROLL = 8
    NACC = 4

    def gather_loop(ids_ref, table_ref, mask):
        def body(i, carry):
            r, c = carry[0], carry[1]
            accs = list(carry[2:])
            for u in range(UNROLL):
                ids = plsc.load_gather(ids_ref, [r, c])
                accs[u % NACC] = accs[u % NACC] + plsc.load_gather(
                    table_ref, [lax.bitwise_and(ids, mask)])
                c = c + LANES
                wrap = c >= HIST
                c = jnp.where(wrap, c - HIST, c)
                r = jnp.where(wrap, r + 1, r)
            return (r, c) + tuple(accs)

        zero = jnp.zeros((LANES,), jnp.float32)
        init = (jnp.zeros((LANES,), jnp.int32), lane) + (zero,) * NACC
        out = lax.fori_loop(0, VECS // UNROLL, body, init)
        accs = out[2:]
        return (accs[0] + accs[1]) + (accs[2] + accs[3])

    cp0.wait()
    acc = gather_loop(ids0_v, acc_v, ZCH0 - 1)
    cp1.wait()
    acc = acc + gather_loop(ids1_v, rs1_v, ZCH1 - 1)
    acc_v[...] = acc
    pltpu.sync_copy(acc_v, out_hbm.at[wid])


def _tc_finish(partials):
    def body(p_ref, o_ref):
        o_ref[0, 0] = jnp.sum(p_ref[...]) * (1.0 / (BATCH * 2 * EMBED))

    return pl.pallas_call(
        body,
        out_shape=jax.ShapeDtypeStruct((1, 1), jnp.float32),
        out_specs=pl.BlockSpec(memory_space=pltpu.SMEM),
    )(partials)


def kernel(ids_0, ids_1, W0, W1):
    partials = _sc_partials(ids_0, ids_1, W0, W1)
    return _tc_finish(partials)[0, 0]


# speedup vs baseline: 22.9644x; 1.0200x over previous
import functools

import jax
import jax.numpy as jnp
from jax import lax
from jax.experimental import pallas as pl
from jax.experimental.pallas import tpu as pltpu
from jax.experimental.pallas import tpu_sc as plsc

ZCH0 = 16
ZCH1 = 32
EMBED = 64
BATCH = 4096
HIST = 20

NC = 2
NS = 16
LANES = 16
NW = NC * NS
ROWS = BATCH // NW
CHUNK = ROWS * HIST
VECS = CHUNK // LANES

_SC_MESH = plsc.VectorSubcoreMesh(core_axis_name="c", subcore_axis_name="s")


@functools.partial(
    pl.kernel,
    out_type=jax.ShapeDtypeStruct((NW, LANES), jnp.float32),
    mesh=_SC_MESH,
    scratch_types=[
        pltpu.VMEM((ROWS, HIST), jnp.int32),
        pltpu.VMEM((ZCH1, EMBED), jnp.float32),
        pltpu.VMEM((ZCH1,), jnp.float32),
        pltpu.VMEM((LANES,), jnp.float32),
        pltpu.SemaphoreType.DMA,
        pltpu.SemaphoreType.DMA,
    ],
    compiler_params=pltpu.CompilerParams(needs_layout_passes=False),
)
def _sc_partials(ids1_hbm, w1_hbm, out_hbm,
                 ids1_v, w1_v, rs1_v, acc_v, sem_i, sem_w):
    wid = lax.axis_index("s") * NC + lax.axis_index("c")
    cp1 = pltpu.async_copy(
        ids1_hbm.at[pl.ds(wid * ROWS, ROWS), :], ids1_v, sem_i)
    cpw = pltpu.async_copy(w1_hbm, w1_v, sem_w)

    lane = lax.iota(jnp.int32, LANES)
    cpw.wait()

    def rowsums(row0):
        rows = lane + row0
        parts = [jnp.zeros((LANES,), jnp.float32) for _ in range(4)]
        for dcol in range(EMBED):
            col = jnp.full((LANES,), dcol, jnp.int32)
            parts[dcol % 4] = parts[dcol % 4] + plsc.load_gather(
                w1_v, [rows, col])
        return (parts[0] + parts[1]) + (parts[2] + parts[3])

    rs1_v[pl.ds(0, LANES)] = rowsums(0)
    rs1_v[pl.ds(LANES, LANES)] = rowsums(LANES)

    UNROLL = 8
    NACC = 4

    def body(i, carry):
        r, c = carry[0], carry[1]
        accs = list(carry[2:])
        for u in range(UNROLL):
            ids = plsc.load_gather(ids1_v, [r, c])
            accs[u % NACC] = accs[u % NACC] + plsc.load_gather(
                rs1_v, [lax.bitwise_and(ids, ZCH1 - 1)])
            c = c + LANES
            wrap = c >= HIST
            c = jnp.where(wrap, c - HIST, c)
            r = jnp.where(wrap, r + 1, r)
        return (r, c) + tuple(accs)

    zero = jnp.zeros((LANES,), jnp.float32)
    init = (jnp.zeros((LANES,), jnp.int32), lane) + (zero,) * NACC
    cp1.wait()
    out = lax.fori_loop(0, VECS // UNROLL, body, init)
    accs = out[2:]
    acc_v[...] = (accs[0] + accs[1]) + (accs[2] + accs[3])
    pltpu.sync_copy(acc_v, out_hbm.at[wid])


def _tc_pool0(ids_0, W0):
    def body(ids_ref, w_ref, o_ref):
        rs = jnp.sum(w_ref[...], axis=1)
        iota = lax.broadcasted_iota(jnp.int32, (ZCH0,), 0)
        vals = [jnp.sum(jnp.where(iota == r, rs, 0.0)) for r in range(ZCH0)]
        ids = ids_ref[...]
        bits = [lax.bitwise_and(ids, 1 << k) != 0 for k in range(4)]
        level = vals
        for k in range(4):
            level = [jnp.where(bits[k], level[2 * j + 1], level[2 * j])
                     for j in range(len(level) // 2)]
        o_ref[0, 0] = jnp.sum(level[0])

    return pl.pallas_call(
        body,
        out_shape=jax.ShapeDtypeStruct((1, 1), jnp.float32),
        out_specs=pl.BlockSpec(memory_space=pltpu.SMEM),
    )(ids_0, W0)


def _tc_finish(partials, p0):
    def body(p_ref, s_ref, o_ref):
        o_ref[0, 0] = (jnp.sum(p_ref[...]) + s_ref[0, 0]) * (
            1.0 / (BATCH * 2 * EMBED))

    return pl.pallas_call(
        body,
        out_shape=jax.ShapeDtypeStruct((1, 1), jnp.float32),
        in_specs=[
            pl.BlockSpec(memory_space=pltpu.VMEM),
            pl.BlockSpec(memory_space=pltpu.SMEM),
        ],
        out_specs=pl.BlockSpec(memory_space=pltpu.SMEM),
    )(partials, p0)


def kernel(ids_0, ids_1, W0, W1):
    partials = _sc_partials(ids_1, W1)
    p0 = _tc_pool0(ids_0, W0)
    return _tc_finish(partials, p0)[0, 0]
